# baseline (speedup 1.0000x reference)
import jax
import jax.numpy as jnp
from jax.experimental import pallas as pl

_B, _N, _C = 16, 128, 128
_FILT = 64
_NH = 32
_OUT = 10
_G = 8


def _mgcn_body(x_ref, A_ref, m_ref,
               w1_ref, b1_ref, w2_ref, b2_ref,
               gw0_ref, gb0_ref, gw1_ref, gb1_ref, gw2_ref, gb2_ref,
               fcw_ref, fcb_ref, out_ref):
    f32 = jnp.float32
    bf16 = jnp.bfloat16
    ri = jax.lax.broadcasted_iota(jnp.int32, (_N, _N), 0)
    ci = jax.lax.broadcasted_iota(jnp.int32, (_N, _N), 1)
    on_diag = ri == ci
    eye = jnp.where(on_diag, 1.0, 0.0).astype(f32)

    x3 = x_ref[...]
    A3 = A_ref[...]
    m3 = m_ref[...]
    x2 = x3.reshape(_G * _N, _C)

    def dot_t(a, w):
        return jax.lax.dot_general(a, w, (((1,), (1,)), ((), ())),
                                   preferred_element_type=f32)

    w1 = w1_ref[...]
    U = dot_t(x2, w1[:, :_C]) + b1_ref[...]
    V = dot_t(x2, w1[:, _C:])
    U3b = U.reshape(_G, _N, _NH).astype(bf16)
    Vt3b = jnp.transpose(V.reshape(_G, _N, _NH), (0, 2, 1)).astype(bf16)
    w2rows = jnp.broadcast_to(jnp.transpose(w2_ref[...]), (_NH, _N)).astype(bf16)
    S_parts = []
    for g in range(_G):
        Ug = U3b[g]
        Vtg = Vt3b[g]
        acc = [None] * 4
        for k in range(_NH):
            t = jax.nn.relu(Ug[:, k:k + 1] + Vtg[k:k + 1, :]) * w2rows[k:k + 1, :]
            a = k % 4
            acc[a] = t if acc[a] is None else acc[a] + t
        Sg = ((acc[0].astype(f32) + acc[1].astype(f32))
              + (acc[2].astype(f32) + acc[3].astype(f32)))
        S_parts.append(Sg)
    S = jnp.stack(S_parts, axis=0)
    Ssym = 0.5 * (S + jnp.transpose(S, (0, 2, 1))) + b2_ref[...]
    A_pred = jnp.where(on_diag, 0.0, jnp.exp(Ssym))

    def norm(Ah):
        d_row = 1.0 / jnp.sqrt(jnp.sum(Ah, axis=1, keepdims=True) + 1e-5)
        d_col = jnp.transpose(d_row, (0, 2, 1))
        return d_col * Ah * d_row

    L0 = norm(A3 + eye)
    L1 = norm(A_pred + eye)

    m_col = jnp.transpose(m3, (0, 2, 1))

    def bmm(a, b):
        return jax.lax.dot_general(a, b, (((2,), (1,)), ((0,), (0,))),
                                   preferred_element_type=f32)

    def layer(h, w_ref, b_ref, d):
        h2 = h.reshape(_G * _N, d)
        w = w_ref[...]
        pa = dot_t(h2, w[:, :d]).reshape(_G, _N, _FILT)
        pb = dot_t(h2, w[:, d:]).reshape(_G, _N, _FILT)
        z = bmm(L0, pa) + bmm(L1, pb) + b_ref[...]
        return jax.nn.relu(z * m_col)

    h = layer(x3, gw0_ref, gb0_ref, _C)
    h = layer(h, gw1_ref, gb1_ref, _FILT)
    h = layer(h, gw2_ref, gb2_ref, _FILT)
    pooled = jnp.max(h, axis=1)
    out_ref[...] = dot_t(pooled, fcw_ref[...]) + fcb_ref[...]


def kernel(x, A, mask, ep_w1, ep_b1, ep_w2, ep_b2,
           g_w0, g_b0, g_w1, g_b1, g_w2, g_b2, fc_w, fc_b):
    args = (x, A, mask.reshape(_B, 1, _N),
            ep_w1, ep_b1.reshape(1, _NH), ep_w2, ep_b2.reshape(1, 1),
            g_w0, g_b0.reshape(1, _FILT), g_w1, g_b1.reshape(1, _FILT),
            g_w2, g_b2.reshape(1, _FILT), fc_w, fc_b.reshape(1, _OUT))

    def full(a):
        nd = a.ndim
        return pl.BlockSpec(a.shape, lambda b, _nd=nd: (0,) * _nd)

    batched = [pl.BlockSpec((_G, _N, _C), lambda b: (b, 0, 0)),
               pl.BlockSpec((_G, _N, _N), lambda b: (b, 0, 0)),
               pl.BlockSpec((_G, 1, _N), lambda b: (b, 0, 0))]
    return pl.pallas_call(
        _mgcn_body,
        grid=(_B // _G,),
        in_specs=batched + [full(a) for a in args[3:]],
        out_specs=pl.BlockSpec((_G, _OUT), lambda b: (b, 0)),
        out_shape=jax.ShapeDtypeStruct((_B, _OUT), jnp.float32),
    )(*args)


# speedup vs baseline: 1.0596x; 1.0596x over previous
import jax
import jax.numpy as jnp
from jax.experimental import pallas as pl

_B, _N, _C = 16, 128, 128
_FILT = 64
_NH = 32
_OUT = 10


def _mgcn_body(x_ref, A_ref, m_ref,
               w1_ref, b1_ref, w2_ref, b2_ref,
               gw0_ref, gb0_ref, gw1_ref, gb1_ref, gw2_ref, gb2_ref,
               fcw_ref, fcb_ref, out_ref):
    f32 = jnp.float32
    bf16 = jnp.bfloat16
    ri = jax.lax.broadcasted_iota(jnp.int32, (_N, _N), 0)
    ci = jax.lax.broadcasted_iota(jnp.int32, (_N, _N), 1)
    on_diag = ri == ci
    eye = jnp.where(on_diag, 1.0, 0.0).astype(f32)

    x3 = x_ref[...]
    A3 = A_ref[...]
    m3 = m_ref[...]
    x2 = x3.reshape(_B * _N, _C)

    def dot_t(a, w):
        return jax.lax.dot_general(a, w, (((1,), (1,)), ((), ())),
                                   preferred_element_type=f32)

    w1 = w1_ref[...]
    U = dot_t(x2, w1[:, :_C]) + b1_ref[...]
    V = dot_t(x2, w1[:, _C:])
    U3b = U.reshape(_B, _N, _NH).astype(bf16)
    Vt3b = jnp.transpose(V.reshape(_B, _N, _NH), (0, 2, 1)).astype(bf16)
    w2rows = jnp.broadcast_to(jnp.transpose(w2_ref[...]), (_NH, _N)).astype(bf16)
    S_parts = []
    for g in range(_B):
        Ug = U3b[g]
        Vtg = Vt3b[g]
        acc = [None] * 4
        for k in range(_NH):
            t = jax.nn.relu(Ug[:, k:k + 1] + Vtg[k:k + 1, :]) * w2rows[k:k + 1, :]
            a = k % 4
            acc[a] = t if acc[a] is None else acc[a] + t
        Sg = ((acc[0].astype(f32) + acc[1].astype(f32))
              + (acc[2].astype(f32) + acc[3].astype(f32)))
        S_parts.append(Sg)
    S = jnp.stack(S_parts, axis=0)
    Ssym = 0.5 * (S + jnp.transpose(S, (0, 2, 1))) + b2_ref[...]
    A_pred = jnp.where(on_diag, 0.0, jnp.exp(Ssym))

    def norm(Ah):
        d_row = 1.0 / jnp.sqrt(jnp.sum(Ah, axis=1, keepdims=True) + 1e-5)
        d_col = jnp.transpose(d_row, (0, 2, 1))
        return d_col * Ah * d_row

    L0 = norm(A3 + eye)
    L1 = norm(A_pred + eye)

    m_col = jnp.transpose(m3, (0, 2, 1))

    def bmm(a, b):
        return jax.lax.dot_general(a, b, (((2,), (1,)), ((0,), (0,))),
                                   preferred_element_type=f32)

    def layer(h, w_ref, b_ref, d):
        h2 = h.reshape(_B * _N, d)
        w = w_ref[...]
        pa = dot_t(h2, w[:, :d]).reshape(_B, _N, _FILT)
        pb = dot_t(h2, w[:, d:]).reshape(_B, _N, _FILT)
        z = bmm(L0, pa) + bmm(L1, pb) + b_ref[...]
        return jax.nn.relu(z * m_col)

    h = layer(x3, gw0_ref, gb0_ref, _C)
    h = layer(h, gw1_ref, gb1_ref, _FILT)
    h = layer(h, gw2_ref, gb2_ref, _FILT)
    pooled = jnp.max(h, axis=1)
    out_ref[...] = dot_t(pooled, fcw_ref[...]) + fcb_ref[...]


def kernel(x, A, mask, ep_w1, ep_b1, ep_w2, ep_b2,
           g_w0, g_b0, g_w1, g_b1, g_w2, g_b2, fc_w, fc_b):
    args = (x, A, mask.reshape(_B, 1, _N),
            ep_w1, ep_b1.reshape(1, _NH), ep_w2, ep_b2.reshape(1, 1),
            g_w0, g_b0.reshape(1, _FILT), g_w1, g_b1.reshape(1, _FILT),
            g_w2, g_b2.reshape(1, _FILT), fc_w, fc_b.reshape(1, _OUT))

    def full(a):
        nd = a.ndim
        return pl.BlockSpec(a.shape, lambda *_, _nd=nd: (0,) * _nd)

    return pl.pallas_call(
        _mgcn_body,
        in_specs=[full(a) for a in args],
        out_specs=pl.BlockSpec((_B, _OUT), lambda *_: (0, 0)),
        out_shape=jax.ShapeDtypeStruct((_B, _OUT), jnp.float32),
    )(*args)
